# Initial kernel scaffold; baseline (speedup 1.0000x reference)
import jax
import jax.numpy as jnp
from jax.experimental import pallas as pl


def kernel(inputs, embedding):
    raise NotImplementedError("write your pallas kernel here")


# speedup vs baseline: 2.7200x; 2.7200x over previous
import functools

import jax
import jax.numpy as jnp
from jax import lax
from jax.experimental import pallas as pl
from jax.experimental.pallas import tpu as pltpu
from jax.experimental.pallas import tpu_sc as plsc

NUM_EMBEDDINGS = 100000
NUM_FEATURES = 128
BATCH = 4096
SEQ = 50

_INFO = plsc.get_sparse_core_info()
_NC = _INFO.num_cores
_NS = _INFO.num_subcores
_NW = _NC * _NS
_TOTAL = BATCH * SEQ
_PER_W = _TOTAL // _NW
_CHUNK = 128
_NCHUNK = _PER_W // _CHUNK


def _make_kernel():
  mesh = plsc.VectorSubcoreMesh(core_axis_name="c", subcore_axis_name="s")

  @functools.partial(
      pl.kernel,
      mesh=mesh,
      out_type=jax.ShapeDtypeStruct((_NW, _NCHUNK, _CHUNK, NUM_FEATURES),
                                    jnp.float32),
      scratch_types=[
          pltpu.VMEM((_NCHUNK, _CHUNK), jnp.int32),
          pltpu.VMEM((_CHUNK, NUM_FEATURES), jnp.float32),
          pltpu.SemaphoreType.DMA,
      ],
  )
  def k(idx_hbm, table_hbm, out_hbm, idx_v, rows_v, sem):
    wid = lax.axis_index("s") * _NC + lax.axis_index("c")
    pltpu.sync_copy(idx_hbm.at[wid], idx_v)

    def body(j, _):
      pltpu.async_copy(table_hbm.at[idx_v.at[j]], rows_v, sem).wait()
      pltpu.sync_copy(rows_v, out_hbm.at[wid].at[j])
      return _

    lax.fori_loop(0, _NCHUNK, body, None)

  return k


_kernel_call = _make_kernel()


def kernel(inputs, embedding):
  idx = inputs.reshape(_NW, _NCHUNK, _CHUNK).astype(jnp.int32)
  out = _kernel_call(idx, embedding)
  return out.reshape(BATCH, SEQ, NUM_FEATURES)


# speedup vs baseline: 3.0407x; 1.1179x over previous
import functools

import jax
import jax.numpy as jnp
from jax import lax
from jax.experimental import pallas as pl
from jax.experimental.pallas import tpu as pltpu
from jax.experimental.pallas import tpu_sc as plsc

NUM_EMBEDDINGS = 100000
NUM_FEATURES = 128
BATCH = 4096
SEQ = 50

_INFO = plsc.get_sparse_core_info()
_NC = _INFO.num_cores
_NS = _INFO.num_subcores
_NW = _NC * _NS
_TOTAL = BATCH * SEQ
_PER_W = _TOTAL // _NW
_CHUNK = 128
_NCHUNK = _PER_W // _CHUNK
_NBUF = 5
_NGROUP = _NCHUNK // _NBUF


def _make_kernel():
  mesh = plsc.VectorSubcoreMesh(core_axis_name="c", subcore_axis_name="s")

  @functools.partial(
      pl.kernel,
      mesh=mesh,
      out_type=jax.ShapeDtypeStruct((_NW, _NCHUNK, _CHUNK, NUM_FEATURES),
                                    jnp.float32),
      scratch_types=[
          pltpu.VMEM((_NCHUNK, _CHUNK), jnp.int32),
          pltpu.VMEM((_NBUF, _CHUNK, NUM_FEATURES), jnp.float32),
          pltpu.SemaphoreType.DMA((_NBUF,)),
          pltpu.SemaphoreType.DMA((_NBUF,)),
      ],
  )
  def k(idx_hbm, table_hbm, out_hbm, idx_v, rows_v, sem_g, sem_s):
    wid = lax.axis_index("s") * _NC + lax.axis_index("c")
    pltpu.sync_copy(idx_hbm.at[wid], idx_v)

    def g_start(j, b):
      pltpu.async_copy(table_hbm.at[idx_v.at[j]], rows_v.at[b], sem_g.at[b])

    def g_wait(b):
      pltpu.make_async_copy(
          table_hbm.at[pl.ds(0, _CHUNK)], rows_v.at[b], sem_g.at[b]).wait()

    def s_start(j, b):
      pltpu.async_copy(rows_v.at[b], out_hbm.at[wid].at[j], sem_s.at[b])

    def s_wait(b):
      pltpu.make_async_copy(
          rows_v.at[b], out_hbm.at[wid].at[0], sem_s.at[b]).wait()

    for b in range(_NBUF):
      g_start(b, b)

    def body(i, _):
      base = i * _NBUF
      for b in range(_NBUF):
        g_wait(b)
        s_start(base + b, b)
      for b in range(_NBUF):
        s_wait(b)

        @pl.when(i < _NGROUP - 1)
        def _():
          g_start(base + _NBUF + b, b)

      return _

    lax.fori_loop(0, _NGROUP, body, None)

  return k


_kernel_call = _make_kernel()


def kernel(inputs, embedding):
  idx = inputs.reshape(_NW, _NCHUNK, _CHUNK).astype(jnp.int32)
  out = _kernel_call(idx, embedding)
  return out.reshape(BATCH, SEQ, NUM_FEATURES)
